# baseline (speedup 1.0000x reference)
import functools

import jax
import jax.numpy as jnp
from jax.experimental import pallas as pl
from jax.experimental.pallas import tpu as pltpu

_NCHUNK = 4
_CHUNK = 256


def _gat_kernel(x_ref, w_ref, a_ref, adj_ref, out_ref, *rest):
    bufs = rest[:_NCHUNK]
    sems = rest[_NCHUNK:]
    copies = []
    for k in range(_NCHUNK):
        cp = pltpu.make_async_copy(
            adj_ref.at[pl.ds(k * _CHUNK, _CHUNK), :], bufs[k], sems[k])
        cp.start()
        copies.append(cp)

    hid = jnp.dot(x_ref[...], w_ref[...], preferred_element_type=jnp.float32)
    d = w_ref.shape[1]
    a1 = a_ref[:d, :]
    a2 = a_ref[d:, :]
    s1 = jnp.dot(hid, a1, preferred_element_type=jnp.float32)
    s2 = jax.lax.dot_general(a2, hid, (((0,), (1,)), ((), ())),
                             preferred_element_type=jnp.float32)

    for k in range(_NCHUNK):
        copies[k].wait()
        s1_blk = s1[k * _CHUNK:(k + 1) * _CHUNK, :]
        logits = s1_blk + s2
        neg = jnp.where(logits >= 0.0, logits, 0.2 * logits)
        e = jnp.where(bufs[k][...] != 0, jnp.exp(-neg), 0.0)
        rowsum = jnp.sum(e, axis=1, keepdims=True)
        h = jnp.dot(e, hid, preferred_element_type=jnp.float32)
        hp = h / (rowsum + 1e-9)
        out_ref[pl.ds(k * _CHUNK, _CHUNK), :] = jnp.where(
            hp > 0.0, hp, jnp.exp(jnp.minimum(hp, 0.0)) - 1.0)


@jax.jit
def kernel(x, adj, W, a):
    n, din = x.shape
    dout = W.shape[1]
    return pl.pallas_call(
        _gat_kernel,
        in_specs=[
            pl.BlockSpec((n, din), lambda: (0, 0)),
            pl.BlockSpec((din, dout), lambda: (0, 0)),
            pl.BlockSpec((2 * dout, 1), lambda: (0, 0)),
            pl.BlockSpec(memory_space=pltpu.MemorySpace.HBM),
        ],
        out_specs=pl.BlockSpec((n, dout), lambda: (0, 0)),
        out_shape=jax.ShapeDtypeStruct((n, dout), jnp.float32),
        scratch_shapes=(
            [pltpu.VMEM((_CHUNK, n), jnp.int32) for _ in range(_NCHUNK)]
            + [pltpu.SemaphoreType.DMA for _ in range(_NCHUNK)]),
    )(x, W, a, adj)


# speedup vs baseline: 1.1312x; 1.1312x over previous
import functools

import jax
import jax.numpy as jnp
from jax.experimental import pallas as pl
from jax.experimental.pallas import tpu as pltpu

_NCHUNK = 4
_CHUNK = 256


def _gat_kernel(x_ref, w_ref, a_ref, *rest):
    adj_refs = rest[:_NCHUNK]
    out_ref = rest[_NCHUNK]

    hid = jnp.dot(x_ref[...], w_ref[...], preferred_element_type=jnp.float32)
    d = w_ref.shape[1]
    a1 = a_ref[:d, :]
    a2 = a_ref[d:, :]
    s1 = jnp.dot(hid, a1, preferred_element_type=jnp.float32)
    s2 = jax.lax.dot_general(a2, hid, (((0,), (1,)), ((), ())),
                             preferred_element_type=jnp.float32)

    for k in range(_NCHUNK):
        s1_blk = s1[k * _CHUNK:(k + 1) * _CHUNK, :]
        logits = s1_blk + s2
        neg = jnp.where(logits >= 0.0, logits, 0.2 * logits)
        e = jnp.where(adj_refs[k][...] != 0, jnp.exp(-neg), 0.0)
        rowsum = jnp.sum(e, axis=1, keepdims=True)
        h = jnp.dot(e, hid, preferred_element_type=jnp.float32)
        hp = h / (rowsum + 1e-9)
        out_ref[pl.ds(k * _CHUNK, _CHUNK), :] = jnp.where(
            hp > 0.0, hp, jnp.exp(jnp.minimum(hp, 0.0)) - 1.0)


@jax.jit
def kernel(x, adj, W, a):
    n, din = x.shape
    dout = W.shape[1]
    adj_specs = [
        pl.BlockSpec((_CHUNK, n), (lambda k: (lambda i: (k, 0)))(k))
        for k in range(_NCHUNK)
    ]
    return pl.pallas_call(
        _gat_kernel,
        grid=(1,),
        in_specs=[
            pl.BlockSpec((n, din), lambda i: (0, 0)),
            pl.BlockSpec((din, dout), lambda i: (0, 0)),
            pl.BlockSpec((2 * dout, 1), lambda i: (0, 0)),
        ] + adj_specs,
        out_specs=pl.BlockSpec((n, dout), lambda i: (0, 0)),
        out_shape=jax.ShapeDtypeStruct((n, dout), jnp.float32),
    )(x, W, a, adj, adj, adj, adj)
